# baseline (speedup 1.0000x reference)
import functools
import math

import jax
import jax.numpy as jnp
from jax import lax
from jax.experimental import pallas as pl
from jax.experimental.pallas import tpu as pltpu
from jax.experimental.pallas import tpu_sc as plsc

_NEGPOS_RATIO = 3.0
_THRESHOLD = 0.5
_NL = 128
_L = 16


def _smooth_l1_sum(d):
    ad = jnp.abs(d)
    return jnp.sum(jnp.where(ad < 1.0, 0.5 * ad * ad, ad - 0.5))


def _match_body(num_anchors, tgt_ref, anc_ref, loc_ref, lab_ref, stats_ref):
    P = num_anchors
    NO = tgt_ref.shape[1]
    NR = anc_ref.shape[1]

    ax = anc_ref[0]
    ay = anc_ref[1]
    aw = anc_ref[2]
    ah = anc_ref[3]
    ax2 = ax + aw
    ay2 = ay + ah
    area_a = (ax2 - ax) * (ay2 - ay)

    i0 = jax.lax.broadcasted_iota(jnp.int32, (NR, _NL), 0)
    i1 = jax.lax.broadcasted_iota(jnp.int32, (NR, _NL), 1)
    flat = i0 * _NL + i1
    valid = flat < P

    tgt = tgt_ref[0]
    tx_v = tgt[:, 0][None, :]
    ty_v = tgt[:, 1][None, :]
    tx2_v = tgt[:, 2][None, :]
    ty2_v = tgt[:, 3][None, :]
    tl_v = tgt[:, 4][None, :]
    tw_v = tx2_v - tx_v
    th_v = ty2_v - ty_v
    log_tw = jnp.log(tw_v)
    log_th = jnp.log(th_v)

    def _sc(a, t):
        return jnp.sum(a[0:1, t:t + 1])

    bto = jnp.full((NR, _NL), -1.0, jnp.float32)
    bti = jnp.zeros((NR, _NL), jnp.int32)
    bpis = []
    for t in range(NO):
        tx, ty, tx2, ty2 = (_sc(tx_v, t), _sc(ty_v, t), _sc(tx2_v, t), _sc(ty2_v, t))
        area_t = (tx2 - tx) * (ty2 - ty)
        w = jnp.clip(jnp.minimum(tx2, ax2) - jnp.maximum(tx, ax), 0.0, None)
        h = jnp.clip(jnp.minimum(ty2, ay2) - jnp.maximum(ty, ay), 0.0, None)
        inter = w * h
        ov = inter / (area_t + area_a - inter)
        upd = ov > bto
        bti = jnp.where(upd, t, bti)
        bto = jnp.where(upd, ov, bto)
        mx = jnp.max(jnp.where(valid, ov, -1.0))
        bpis.append(jnp.min(jnp.where((ov == mx) & valid, flat, P)))

    scat = [jnp.zeros((NR, _NL), jnp.float32) for _ in range(5)]
    for t in range(NO):
        m = flat == bpis[t]
        pr_x = jnp.sum(jnp.where(m, ax, 0.0))
        pr_y = jnp.sum(jnp.where(m, ay, 0.0))
        pr_w = jnp.sum(jnp.where(m, aw, 0.0))
        pr_h = jnp.sum(jnp.where(m, ah, 0.0))
        log_pr_w = jnp.sum(jnp.where(m, jnp.log(jnp.where(m, aw, 1.0)), 0.0))
        log_pr_h = jnp.sum(jnp.where(m, jnp.log(jnp.where(m, ah, 1.0)), 0.0))
        f = [(_sc(tx_v, t) - pr_x) / pr_w,
             (_sc(ty_v, t) - pr_y) / pr_h,
             _sc(log_tw, t) - log_pr_w,
             _sc(log_th, t) - log_pr_h,
             _sc(tl_v, t)]
        scat = [jnp.where(m, f[j], scat[j]) for j in range(5)]

    g = [jnp.zeros((NR, _NL), jnp.float32) for _ in range(5)]
    for t in range(NO):
        tm = bti == t
        vals = [_sc(tx_v, t), _sc(ty_v, t), _sc(tw_v, t), _sc(th_v, t), _sc(tl_v, t)]
        g = [jnp.where(tm, vals[j], g[j]) for j in range(5)]

    second = [(g[0] - ax) / aw, (g[1] - ay) / ah,
              jnp.log(g[2]) - jnp.log(aw), jnp.log(g[3]) - jnp.log(ah)]

    over = bto > _THRESHOLD
    lab = jnp.where(over, g[4], scat[4])
    lab = jnp.where(valid, lab, 0.0)
    pos = lab > 0.0

    locsum = jnp.float32(0.0)
    for j in range(4):
        locT_j = jnp.where(over, second[j], scat[j])
        locsum = locsum + _smooth_l1_sum(
            jnp.where(pos, loc_ref[0, j] - locT_j, 0.0))
    npos = jnp.sum(jnp.where(pos, 1.0, 0.0))

    lab_ref[0] = lab
    iota8 = jax.lax.broadcasted_iota(jnp.int32, (8, 1), 0)
    stats_ref[0] = jnp.where(iota8 == 0, npos,
                             jnp.where(iota8 == 1, locsum, 0.0))


def _conf_body(num_anchors, chunk, conf_ref, lab_ref, cls_ref, stats_ref):
    pc = pl.program_id(1)
    Pc = conf_ref.shape[1]
    C = conf_ref.shape[2]

    rows = pc * chunk + jax.lax.broadcasted_iota(jnp.int32, (Pc, 1), 0)
    valid = rows < num_anchors

    cf = conf_ref[0]
    lab = lab_ref[0]
    pos = valid & (lab > 0.0)
    posf = pos.astype(jnp.float32)

    cmax = jnp.max(cf, axis=1, keepdims=True)
    lse = jnp.log(jnp.sum(jnp.exp(cf - cmax), axis=1, keepdims=True)) + cmax
    c0 = cf[:, 0:1]
    iota_c = jax.lax.broadcasted_iota(jnp.int32, (Pc, C), 1)
    clab = jnp.sum(jnp.where(iota_c == lab.astype(jnp.int32), cf, 0.0),
                   axis=1, keepdims=True)

    cls_ref[0] = jnp.where(valid, (lse - c0) * (1.0 - posf), 0.0)
    possum = jnp.sum(jnp.where(pos, lse - clab, 0.0))

    iota8 = jax.lax.broadcasted_iota(jnp.int32, (8, 1), 0)
    col = jnp.where(iota8 == 0, possum, 0.0)

    @pl.when(pc == 0)
    def _():
        stats_ref[0] = col

    @pl.when(pc != 0)
    def _():
        stats_ref[0] = stats_ref[0] + col


def _sc_mine_body(num_anchors, num_classes, cls_hbm, bits_hbm, statm_hbm,
                  statc_hbm, out_hbm, row_v, rowi_v, stm_v, stc_v, outv, perm_v):
    P = num_anchors
    b = lax.axis_index("s") * 2 + lax.axis_index("c")
    pltpu.sync_copy(cls_hbm.at[b], row_v)
    pltpu.sync_copy(bits_hbm.at[b], rowi_v)
    pltpu.sync_copy(statm_hbm.at[b], stm_v)
    pltpu.sync_copy(statc_hbm.at[b], stc_v)

    iota = lax.broadcasted_iota(jnp.int32, (_L,), 0)
    lane0 = iota == 0
    zeros = jnp.zeros((_L,), jnp.float32)
    ones = jnp.ones((_L,), jnp.float32)

    def _lanetree(x, op):
        for sh in (8, 4, 2, 1):
            perm_v[pl.ds(0, _L)] = x
            perm_v[pl.ds(_L, _L)] = x
            x = op(x, perm_v[pl.ds(sh, _L)])
        return x

    npos = _lanetree(jnp.where(lane0, stm_v[...], zeros), jnp.add)
    k = jnp.minimum(_NEGPOS_RATIO * npos, float(P - 1))

    ngrp = row_v.shape[0] // (8 * _L)

    def count_ge(cand):
        def grp(gi, acc):
            for u in range(8):
                bits = rowi_v[pl.ds(gi * (8 * _L) + u * _L, _L)]
                acc = acc + jnp.where(bits >= cand, ones, zeros)
            return acc
        return _lanetree(lax.fori_loop(0, ngrp, grp, zeros), jnp.add)

    t = jnp.zeros((_L,), jnp.int32)
    for it in range(31):
        cand = t | jnp.full((_L,), 1 << (30 - it), jnp.int32)
        t = jnp.where(count_ge(cand) >= k, cand, t)

    def tail(gi, carry):
        sumv, cntv, vmaxv = carry
        for u in range(8):
            base = gi * (8 * _L) + u * _L
            bits = rowi_v[pl.ds(base, _L)]
            vals = row_v[pl.ds(base, _L)]
            gt = bits > t
            sumv = sumv + jnp.where(gt, vals, zeros)
            cntv = cntv + jnp.where(gt, ones, zeros)
            vmaxv = jnp.maximum(vmaxv, jnp.where(bits <= t, vals, zeros))
        return (sumv, cntv, vmaxv)

    sumv, cntv, vmaxv = lax.fori_loop(0, ngrp, tail, (zeros, zeros, zeros))
    sumgt = _lanetree(sumv, jnp.add)
    m = k - _lanetree(cntv, jnp.add)
    vf = _lanetree(vmaxv, jnp.maximum)
    negsum = sumgt + jnp.where(m > 0.0, m * vf, zeros)

    clsloss = (stc_v[...] + negsum
               + (float(P) - (npos + k)) * math.log(num_classes))
    outv[...] = jnp.where(lane0, clsloss, zeros)
    pltpu.sync_copy(outv, out_hbm.at[b])


def _final_body(cls_part_ref, statm_ref, out_ref):
    clsloss = cls_part_ref[...][:, 0:1]
    npos = statm_ref[...][:, 0:1]
    locsum = statm_ref[...][:, 1:2]
    N = jnp.sum(npos)
    out0 = jnp.sum(locsum) / N
    out1 = jnp.sum(clsloss) / N
    iota_s = jax.lax.broadcasted_iota(jnp.int32, (1, 128), 1)
    out_ref[...] = jnp.where(iota_s == 0, out0, jnp.where(iota_s == 1, out1, 0.0))


def kernel(loc_pred, conf_pred, targets, anchors):
    B, P, C = conf_pred.shape
    NO = targets.shape[1]
    NR = (P + _NL - 1) // _NL
    PAD = NR * _NL

    anc = jnp.pad(anchors.T, ((0, 0), (0, PAD - P))).reshape(4, NR, _NL)
    loc4 = jnp.pad(jnp.transpose(loc_pred, (0, 2, 1)),
                   ((0, 0), (0, 0), (0, PAD - P))).reshape(B, 4, NR, _NL)

    lab_t, stats_m = pl.pallas_call(
        functools.partial(_match_body, P),
        grid=(B,),
        in_specs=[
            pl.BlockSpec((1, NO, 5), lambda b: (b, 0, 0)),
            pl.BlockSpec((4, NR, _NL), lambda b: (0, 0, 0)),
            pl.BlockSpec((1, 4, NR, _NL), lambda b: (b, 0, 0, 0)),
        ],
        out_specs=[
            pl.BlockSpec((1, NR, _NL), lambda b: (b, 0, 0)),
            pl.BlockSpec((1, 8, 1), lambda b: (b, 0, 0)),
        ],
        out_shape=[
            jax.ShapeDtypeStruct((B, NR, _NL), jnp.float32),
            jax.ShapeDtypeStruct((B, 8, 1), jnp.float32),
        ],
    )(targets, anc, loc4)

    lab = lab_t.reshape(B, PAD)[:, :P].reshape(B, P, 1)

    chunk = 1096
    nch = (P + chunk - 1) // chunk
    cls, stats_c = pl.pallas_call(
        functools.partial(_conf_body, P, chunk),
        grid=(B, nch),
        in_specs=[
            pl.BlockSpec((1, chunk, C), lambda b, c: (b, c, 0)),
            pl.BlockSpec((1, chunk, 1), lambda b, c: (b, c, 0)),
        ],
        out_specs=[
            pl.BlockSpec((1, chunk, 1), lambda b, c: (b, c, 0)),
            pl.BlockSpec((1, 8, 1), lambda b, c: (b, 0, 0)),
        ],
        out_shape=[
            jax.ShapeDtypeStruct((B, P, 1), jnp.float32),
            jax.ShapeDtypeStruct((B, 8, 1), jnp.float32),
        ],
    )(conf_pred, lab)

    Ppad = ((P + 127) // 128) * 128
    cls_p = jnp.pad(cls.reshape(B, P), ((0, 0), (0, Ppad - P)))
    statm16 = jnp.pad(stats_m.reshape(B, 8), ((0, 0), (0, 8)))
    statc16 = jnp.pad(stats_c.reshape(B, 8), ((0, 0), (0, 8)))

    mesh = plsc.VectorSubcoreMesh(core_axis_name="c", subcore_axis_name="s")
    sc_mine = functools.partial(
        pl.kernel,
        mesh=mesh,
        out_type=jax.ShapeDtypeStruct((B, _L), jnp.float32),
        scratch_types=[
            pltpu.VMEM((Ppad,), jnp.float32),
            pltpu.VMEM((Ppad,), jnp.int32),
            pltpu.VMEM((_L,), jnp.float32),
            pltpu.VMEM((_L,), jnp.float32),
            pltpu.VMEM((_L,), jnp.float32),
            pltpu.VMEM((2 * _L,), jnp.float32),
        ],
    )(functools.partial(_sc_mine_body, P, C))
    cls_bits = jax.lax.bitcast_convert_type(cls_p, jnp.int32)
    cls_part = sc_mine(cls_p, cls_bits, statm16, statc16)

    out = pl.pallas_call(
        _final_body,
        grid=(1,),
        in_specs=[
            pl.BlockSpec((B, _L), lambda i: (0, 0)),
            pl.BlockSpec((B, 8), lambda i: (0, 0)),
        ],
        out_specs=pl.BlockSpec((1, 128), lambda i: (0, 0)),
        out_shape=jax.ShapeDtypeStruct((1, 128), jnp.float32),
    )(cls_part, stats_m.reshape(B, 8))

    return out[0, 0], out[0, 1]


# speedup vs baseline: 1.1762x; 1.1762x over previous
import functools
import math

import jax
import jax.numpy as jnp
from jax import lax
from jax.experimental import pallas as pl
from jax.experimental.pallas import tpu as pltpu
from jax.experimental.pallas import tpu_sc as plsc

_NEGPOS_RATIO = 3.0
_THRESHOLD = 0.5
_NL = 128
_L = 16


def _smooth_l1_sum(d):
    ad = jnp.abs(d)
    return jnp.sum(jnp.where(ad < 1.0, 0.5 * ad * ad, ad - 0.5))


def _match_body(num_anchors, tgt_ref, anc_ref, loc_ref, lab_ref, stats_ref):
    P = num_anchors
    NO = tgt_ref.shape[1]
    NR = anc_ref.shape[1]

    ax = anc_ref[0]
    ay = anc_ref[1]
    aw = anc_ref[2]
    ah = anc_ref[3]
    ax2 = ax + aw
    ay2 = ay + ah
    area_a = (ax2 - ax) * (ay2 - ay)

    i0 = jax.lax.broadcasted_iota(jnp.int32, (NR, _NL), 0)
    i1 = jax.lax.broadcasted_iota(jnp.int32, (NR, _NL), 1)
    flat = i0 * _NL + i1
    valid = flat < P

    tgt = tgt_ref[0]
    tx_v = tgt[:, 0][None, :]
    ty_v = tgt[:, 1][None, :]
    tx2_v = tgt[:, 2][None, :]
    ty2_v = tgt[:, 3][None, :]
    tl_v = tgt[:, 4][None, :]
    tw_v = tx2_v - tx_v
    th_v = ty2_v - ty_v
    log_tw = jnp.log(tw_v)
    log_th = jnp.log(th_v)

    def _sc(a, t):
        return jnp.sum(a[0:1, t:t + 1])

    bto = jnp.full((NR, _NL), -1.0, jnp.float32)
    bti = jnp.zeros((NR, _NL), jnp.int32)
    bpis = []
    for t in range(NO):
        tx, ty, tx2, ty2 = (_sc(tx_v, t), _sc(ty_v, t), _sc(tx2_v, t), _sc(ty2_v, t))
        area_t = (tx2 - tx) * (ty2 - ty)
        w = jnp.clip(jnp.minimum(tx2, ax2) - jnp.maximum(tx, ax), 0.0, None)
        h = jnp.clip(jnp.minimum(ty2, ay2) - jnp.maximum(ty, ay), 0.0, None)
        inter = w * h
        ov = inter / (area_t + area_a - inter)
        upd = ov > bto
        bti = jnp.where(upd, t, bti)
        bto = jnp.where(upd, ov, bto)
        mx = jnp.max(jnp.where(valid, ov, -1.0))
        bpis.append(jnp.min(jnp.where((ov == mx) & valid, flat, P)))

    scat = [jnp.zeros((NR, _NL), jnp.float32) for _ in range(5)]
    for t in range(NO):
        m = flat == bpis[t]
        pr_x = jnp.sum(jnp.where(m, ax, 0.0))
        pr_y = jnp.sum(jnp.where(m, ay, 0.0))
        pr_w = jnp.sum(jnp.where(m, aw, 0.0))
        pr_h = jnp.sum(jnp.where(m, ah, 0.0))
        log_pr_w = jnp.sum(jnp.where(m, jnp.log(jnp.where(m, aw, 1.0)), 0.0))
        log_pr_h = jnp.sum(jnp.where(m, jnp.log(jnp.where(m, ah, 1.0)), 0.0))
        f = [(_sc(tx_v, t) - pr_x) / pr_w,
             (_sc(ty_v, t) - pr_y) / pr_h,
             _sc(log_tw, t) - log_pr_w,
             _sc(log_th, t) - log_pr_h,
             _sc(tl_v, t)]
        scat = [jnp.where(m, f[j], scat[j]) for j in range(5)]

    g = [jnp.zeros((NR, _NL), jnp.float32) for _ in range(5)]
    for t in range(NO):
        tm = bti == t
        vals = [_sc(tx_v, t), _sc(ty_v, t), _sc(tw_v, t), _sc(th_v, t), _sc(tl_v, t)]
        g = [jnp.where(tm, vals[j], g[j]) for j in range(5)]

    second = [(g[0] - ax) / aw, (g[1] - ay) / ah,
              jnp.log(g[2]) - jnp.log(aw), jnp.log(g[3]) - jnp.log(ah)]

    over = bto > _THRESHOLD
    lab = jnp.where(over, g[4], scat[4])
    lab = jnp.where(valid, lab, 0.0)
    pos = lab > 0.0

    locsum = jnp.float32(0.0)
    for j in range(4):
        locT_j = jnp.where(over, second[j], scat[j])
        locsum = locsum + _smooth_l1_sum(
            jnp.where(pos, loc_ref[0, j] - locT_j, 0.0))
    npos = jnp.sum(jnp.where(pos, 1.0, 0.0))

    lab_ref[0] = lab
    iota8 = jax.lax.broadcasted_iota(jnp.int32, (8, 1), 0)
    stats_ref[0] = jnp.where(iota8 == 0, npos,
                             jnp.where(iota8 == 1, locsum, 0.0))


def _conf_body(num_anchors, chunk, conf_ref, lab_ref, cls_ref, stats_ref):
    pc = pl.program_id(1)
    Pc = conf_ref.shape[1]
    C = conf_ref.shape[2]

    rows = pc * chunk + jax.lax.broadcasted_iota(jnp.int32, (Pc, 1), 0)
    valid = rows < num_anchors

    cf = conf_ref[0]
    lab = lab_ref[0]
    pos = valid & (lab > 0.0)
    posf = pos.astype(jnp.float32)

    cmax = jnp.max(cf, axis=1, keepdims=True)
    lse = jnp.log(jnp.sum(jnp.exp(cf - cmax), axis=1, keepdims=True)) + cmax
    c0 = cf[:, 0:1]
    iota_c = jax.lax.broadcasted_iota(jnp.int32, (Pc, C), 1)
    clab = jnp.sum(jnp.where(iota_c == lab.astype(jnp.int32), cf, 0.0),
                   axis=1, keepdims=True)

    cls_ref[0] = jnp.where(valid, (lse - c0) * (1.0 - posf), 0.0)
    possum = jnp.sum(jnp.where(pos, lse - clab, 0.0))

    iota8 = jax.lax.broadcasted_iota(jnp.int32, (8, 1), 0)
    col = jnp.where(iota8 == 0, possum, 0.0)

    @pl.when(pc == 0)
    def _():
        stats_ref[0] = col

    @pl.when(pc != 0)
    def _():
        stats_ref[0] = stats_ref[0] + col


def _sc_mine_body(num_anchors, num_classes, cls_hbm, bits_hbm, statm_hbm,
                  statc_hbm, out_hbm, row_v, rowi_v, stm_v, stc_v, outv, perm_v):
    P = num_anchors
    b = lax.axis_index("s") * 2 + lax.axis_index("c")
    pltpu.sync_copy(cls_hbm.at[b], row_v)
    pltpu.sync_copy(bits_hbm.at[b], rowi_v)
    pltpu.sync_copy(statm_hbm.at[b], stm_v)
    pltpu.sync_copy(statc_hbm.at[b], stc_v)

    iota = lax.broadcasted_iota(jnp.int32, (_L,), 0)
    lane0 = iota == 0
    zeros = jnp.zeros((_L,), jnp.float32)
    ones = jnp.ones((_L,), jnp.float32)

    def _lanetree(x, op):
        for sh in (8, 4, 2, 1):
            perm_v[pl.ds(0, _L)] = x
            perm_v[pl.ds(_L, _L)] = x
            x = op(x, perm_v[pl.ds(sh, _L)])
        return x

    npos = _lanetree(jnp.where(lane0, stm_v[...], zeros), jnp.add)
    k = jnp.minimum(_NEGPOS_RATIO * npos, float(P - 1))

    ngrp = row_v.shape[0] // (8 * _L)

    def count_ge(cand):
        def grp(gi, acc):
            for u in range(8):
                bits = rowi_v[pl.ds(gi * (8 * _L) + u * _L, _L)]
                acc = acc + jnp.where(bits >= cand, ones, zeros)
            return acc
        return _lanetree(lax.fori_loop(0, ngrp, grp, zeros), jnp.add)

    t = jnp.zeros((_L,), jnp.int32)
    for it in range(31):
        cand = t | jnp.full((_L,), 1 << (30 - it), jnp.int32)
        t = jnp.where(count_ge(cand) >= k, cand, t)

    def tail(gi, carry):
        sumv, cntv, vmaxv = carry
        for u in range(8):
            base = gi * (8 * _L) + u * _L
            bits = rowi_v[pl.ds(base, _L)]
            vals = row_v[pl.ds(base, _L)]
            gt = bits > t
            sumv = sumv + jnp.where(gt, vals, zeros)
            cntv = cntv + jnp.where(gt, ones, zeros)
            vmaxv = jnp.maximum(vmaxv, jnp.where(bits <= t, vals, zeros))
        return (sumv, cntv, vmaxv)

    sumv, cntv, vmaxv = lax.fori_loop(0, ngrp, tail, (zeros, zeros, zeros))
    sumgt = _lanetree(sumv, jnp.add)
    m = k - _lanetree(cntv, jnp.add)
    vf = _lanetree(vmaxv, jnp.maximum)
    negsum = sumgt + jnp.where(m > 0.0, m * vf, zeros)

    clsloss = (stc_v[...] + negsum
               + (float(P) - (npos + k)) * math.log(num_classes))
    outv[...] = jnp.where(lane0, clsloss, zeros)
    pltpu.sync_copy(outv, out_hbm.at[b])


def _final_body(cls_part_ref, statm_ref, out_ref):
    clsloss = cls_part_ref[...][:, 0:1]
    npos = statm_ref[...][:, 0:1]
    locsum = statm_ref[...][:, 1:2]
    N = jnp.sum(npos)
    out0 = jnp.sum(locsum) / N
    out1 = jnp.sum(clsloss) / N
    iota_s = jax.lax.broadcasted_iota(jnp.int32, (1, 128), 1)
    out_ref[...] = jnp.where(iota_s == 0, out0, jnp.where(iota_s == 1, out1, 0.0))


def kernel(loc_pred, conf_pred, targets, anchors):
    B, P, C = conf_pred.shape
    NO = targets.shape[1]
    NR = (P + _NL - 1) // _NL
    PAD = NR * _NL

    anc = jnp.pad(anchors.T, ((0, 0), (0, PAD - P))).reshape(4, NR, _NL)
    loc4 = jnp.pad(jnp.transpose(loc_pred, (0, 2, 1)),
                   ((0, 0), (0, 0), (0, PAD - P))).reshape(B, 4, NR, _NL)

    lab_t, stats_m = pl.pallas_call(
        functools.partial(_match_body, P),
        grid=(B,),
        in_specs=[
            pl.BlockSpec((1, NO, 5), lambda b: (b, 0, 0)),
            pl.BlockSpec((4, NR, _NL), lambda b: (0, 0, 0)),
            pl.BlockSpec((1, 4, NR, _NL), lambda b: (b, 0, 0, 0)),
        ],
        out_specs=[
            pl.BlockSpec((1, NR, _NL), lambda b: (b, 0, 0)),
            pl.BlockSpec((1, 8, 1), lambda b: (b, 0, 0)),
        ],
        out_shape=[
            jax.ShapeDtypeStruct((B, NR, _NL), jnp.float32),
            jax.ShapeDtypeStruct((B, 8, 1), jnp.float32),
        ],
    )(targets, anc, loc4)

    lab = lab_t.reshape(B, PAD)[:, :P].reshape(B, P, 1)

    chunk = 4368
    nch = (P + chunk - 1) // chunk
    cls, stats_c = pl.pallas_call(
        functools.partial(_conf_body, P, chunk),
        grid=(B, nch),
        in_specs=[
            pl.BlockSpec((1, chunk, C), lambda b, c: (b, c, 0)),
            pl.BlockSpec((1, chunk, 1), lambda b, c: (b, c, 0)),
        ],
        out_specs=[
            pl.BlockSpec((1, chunk, 1), lambda b, c: (b, c, 0)),
            pl.BlockSpec((1, 8, 1), lambda b, c: (b, 0, 0)),
        ],
        out_shape=[
            jax.ShapeDtypeStruct((B, P, 1), jnp.float32),
            jax.ShapeDtypeStruct((B, 8, 1), jnp.float32),
        ],
    )(conf_pred, lab)

    Ppad = ((P + 127) // 128) * 128
    cls_p = jnp.pad(cls.reshape(B, P), ((0, 0), (0, Ppad - P)))
    statm16 = jnp.pad(stats_m.reshape(B, 8), ((0, 0), (0, 8)))
    statc16 = jnp.pad(stats_c.reshape(B, 8), ((0, 0), (0, 8)))

    mesh = plsc.VectorSubcoreMesh(core_axis_name="c", subcore_axis_name="s")
    sc_mine = functools.partial(
        pl.kernel,
        mesh=mesh,
        out_type=jax.ShapeDtypeStruct((B, _L), jnp.float32),
        scratch_types=[
            pltpu.VMEM((Ppad,), jnp.float32),
            pltpu.VMEM((Ppad,), jnp.int32),
            pltpu.VMEM((_L,), jnp.float32),
            pltpu.VMEM((_L,), jnp.float32),
            pltpu.VMEM((_L,), jnp.float32),
            pltpu.VMEM((2 * _L,), jnp.float32),
        ],
    )(functools.partial(_sc_mine_body, P, C))
    cls_bits = jax.lax.bitcast_convert_type(cls_p, jnp.int32)
    cls_part = sc_mine(cls_p, cls_bits, statm16, statc16)

    out = pl.pallas_call(
        _final_body,
        grid=(1,),
        in_specs=[
            pl.BlockSpec((B, _L), lambda i: (0, 0)),
            pl.BlockSpec((B, 8), lambda i: (0, 0)),
        ],
        out_specs=pl.BlockSpec((1, 128), lambda i: (0, 0)),
        out_shape=jax.ShapeDtypeStruct((1, 128), jnp.float32),
    )(cls_part, stats_m.reshape(B, 8))

    return out[0, 0], out[0, 1]


# speedup vs baseline: 1.2110x; 1.0296x over previous
import functools
import math

import jax
import jax.numpy as jnp
from jax import lax
from jax.experimental import pallas as pl
from jax.experimental.pallas import tpu as pltpu
from jax.experimental.pallas import tpu_sc as plsc

_NEGPOS_RATIO = 3.0
_THRESHOLD = 0.5
_NL = 128
_L = 16


def _smooth_l1_sum(d):
    ad = jnp.abs(d)
    return jnp.sum(jnp.where(ad < 1.0, 0.5 * ad * ad, ad - 0.5))


def _match_body(num_anchors, tgt_ref, anc_ref, loc_ref, lab_ref, stats_ref):
    P = num_anchors
    NO = tgt_ref.shape[1]
    NR = anc_ref.shape[1]

    ax = anc_ref[0]
    ay = anc_ref[1]
    aw = anc_ref[2]
    ah = anc_ref[3]
    ax2 = ax + aw
    ay2 = ay + ah
    area_a = (ax2 - ax) * (ay2 - ay)

    i0 = jax.lax.broadcasted_iota(jnp.int32, (NR, _NL), 0)
    i1 = jax.lax.broadcasted_iota(jnp.int32, (NR, _NL), 1)
    flat = i0 * _NL + i1
    valid = flat < P

    tgt = tgt_ref[0]
    tx_v = tgt[:, 0][None, :]
    ty_v = tgt[:, 1][None, :]
    tx2_v = tgt[:, 2][None, :]
    ty2_v = tgt[:, 3][None, :]
    tl_v = tgt[:, 4][None, :]
    tw_v = tx2_v - tx_v
    th_v = ty2_v - ty_v
    log_tw = jnp.log(tw_v)
    log_th = jnp.log(th_v)

    def _sc(a, t):
        return jnp.sum(a[0:1, t:t + 1])

    bto = jnp.full((NR, _NL), -1.0, jnp.float32)
    bti = jnp.zeros((NR, _NL), jnp.int32)
    bpis = []
    for t in range(NO):
        tx, ty, tx2, ty2 = (_sc(tx_v, t), _sc(ty_v, t), _sc(tx2_v, t), _sc(ty2_v, t))
        area_t = (tx2 - tx) * (ty2 - ty)
        w = jnp.clip(jnp.minimum(tx2, ax2) - jnp.maximum(tx, ax), 0.0, None)
        h = jnp.clip(jnp.minimum(ty2, ay2) - jnp.maximum(ty, ay), 0.0, None)
        inter = w * h
        ov = inter / (area_t + area_a - inter)
        upd = ov > bto
        bti = jnp.where(upd, t, bti)
        bto = jnp.where(upd, ov, bto)
        mx = jnp.max(jnp.where(valid, ov, -1.0))
        bpis.append(jnp.min(jnp.where((ov == mx) & valid, flat, P)))

    log_aw = jnp.log(aw)
    log_ah = jnp.log(ah)
    scat = [jnp.zeros((NR, _NL), jnp.float32) for _ in range(5)]
    for t in range(NO):
        m = flat == bpis[t]
        f = [(_sc(tx_v, t) - ax) / aw,
             (_sc(ty_v, t) - ay) / ah,
             _sc(log_tw, t) - log_aw,
             _sc(log_th, t) - log_ah,
             jnp.broadcast_to(_sc(tl_v, t), (NR, _NL))]
        scat = [jnp.where(m, f[j], scat[j]) for j in range(5)]

    g = [jnp.zeros((NR, _NL), jnp.float32) for _ in range(5)]
    for t in range(NO):
        tm = bti == t
        vals = [_sc(tx_v, t), _sc(ty_v, t), _sc(tw_v, t), _sc(th_v, t), _sc(tl_v, t)]
        g = [jnp.where(tm, vals[j], g[j]) for j in range(5)]

    second = [(g[0] - ax) / aw, (g[1] - ay) / ah,
              jnp.log(g[2]) - log_aw, jnp.log(g[3]) - log_ah]

    over = bto > _THRESHOLD
    lab = jnp.where(over, g[4], scat[4])
    lab = jnp.where(valid, lab, 0.0)
    pos = lab > 0.0

    locsum = jnp.float32(0.0)
    for j in range(4):
        locT_j = jnp.where(over, second[j], scat[j])
        locsum = locsum + _smooth_l1_sum(
            jnp.where(pos, loc_ref[0, j] - locT_j, 0.0))
    npos = jnp.sum(jnp.where(pos, 1.0, 0.0))

    lab_ref[0] = lab
    iota8 = jax.lax.broadcasted_iota(jnp.int32, (8, 1), 0)
    stats_ref[0] = jnp.where(iota8 == 0, npos,
                             jnp.where(iota8 == 1, locsum, 0.0))


def _conf_body(num_anchors, chunk, conf_ref, lab_ref, cls_ref, stats_ref):
    pc = pl.program_id(1)
    Pc = conf_ref.shape[1]
    C = conf_ref.shape[2]

    rows = pc * chunk + jax.lax.broadcasted_iota(jnp.int32, (Pc, 1), 0)
    valid = rows < num_anchors

    cf = conf_ref[0]
    lab = lab_ref[0]
    pos = valid & (lab > 0.0)
    posf = pos.astype(jnp.float32)

    cmax = jnp.max(cf, axis=1, keepdims=True)
    lse = jnp.log(jnp.sum(jnp.exp(cf - cmax), axis=1, keepdims=True)) + cmax
    c0 = cf[:, 0:1]
    iota_c = jax.lax.broadcasted_iota(jnp.int32, (Pc, C), 1)
    clab = jnp.sum(jnp.where(iota_c == lab.astype(jnp.int32), cf, 0.0),
                   axis=1, keepdims=True)

    cls_ref[0] = jnp.where(valid, (lse - c0) * (1.0 - posf), 0.0)
    possum = jnp.sum(jnp.where(pos, lse - clab, 0.0))

    iota8 = jax.lax.broadcasted_iota(jnp.int32, (8, 1), 0)
    col = jnp.where(iota8 == 0, possum, 0.0)

    @pl.when(pc == 0)
    def _():
        stats_ref[0] = col

    @pl.when(pc != 0)
    def _():
        stats_ref[0] = stats_ref[0] + col


def _sc_mine_body(num_anchors, num_classes, cls_hbm, bits_hbm, statm_hbm,
                  statc_hbm, out_hbm, row_v, rowi_v, stm_v, stc_v, outv, perm_v):
    P = num_anchors
    b = lax.axis_index("s") * 2 + lax.axis_index("c")
    pltpu.sync_copy(cls_hbm.at[b], row_v)
    pltpu.sync_copy(bits_hbm.at[b], rowi_v)
    pltpu.sync_copy(statm_hbm.at[b], stm_v)
    pltpu.sync_copy(statc_hbm.at[b], stc_v)

    iota = lax.broadcasted_iota(jnp.int32, (_L,), 0)
    lane0 = iota == 0
    zeros = jnp.zeros((_L,), jnp.float32)
    ones = jnp.ones((_L,), jnp.float32)

    def _lanetree(x, op):
        for sh in (8, 4, 2, 1):
            perm_v[pl.ds(0, _L)] = x
            perm_v[pl.ds(_L, _L)] = x
            x = op(x, perm_v[pl.ds(sh, _L)])
        return x

    npos = _lanetree(jnp.where(lane0, stm_v[...], zeros), jnp.add)
    k = jnp.minimum(_NEGPOS_RATIO * npos, float(P - 1))

    ngrp = row_v.shape[0] // (8 * _L)

    def count_ge(cand):
        def grp(gi, acc):
            for u in range(8):
                bits = rowi_v[pl.ds(gi * (8 * _L) + u * _L, _L)]
                acc = acc + jnp.where(bits >= cand, ones, zeros)
            return acc
        return _lanetree(lax.fori_loop(0, ngrp, grp, zeros), jnp.add)

    t = jnp.zeros((_L,), jnp.int32)
    for it in range(31):
        cand = t | jnp.full((_L,), 1 << (30 - it), jnp.int32)
        t = jnp.where(count_ge(cand) >= k, cand, t)

    def tail(gi, carry):
        sumv, cntv, vmaxv = carry
        for u in range(8):
            base = gi * (8 * _L) + u * _L
            bits = rowi_v[pl.ds(base, _L)]
            vals = row_v[pl.ds(base, _L)]
            gt = bits > t
            sumv = sumv + jnp.where(gt, vals, zeros)
            cntv = cntv + jnp.where(gt, ones, zeros)
            vmaxv = jnp.maximum(vmaxv, jnp.where(bits <= t, vals, zeros))
        return (sumv, cntv, vmaxv)

    sumv, cntv, vmaxv = lax.fori_loop(0, ngrp, tail, (zeros, zeros, zeros))
    sumgt = _lanetree(sumv, jnp.add)
    m = k - _lanetree(cntv, jnp.add)
    vf = _lanetree(vmaxv, jnp.maximum)
    negsum = sumgt + jnp.where(m > 0.0, m * vf, zeros)

    clsloss = (stc_v[...] + negsum
               + (float(P) - (npos + k)) * math.log(num_classes))
    outv[...] = jnp.where(lane0, clsloss, zeros)
    pltpu.sync_copy(outv, out_hbm.at[b])


def _final_body(cls_part_ref, statm_ref, out_ref):
    clsloss = cls_part_ref[...][:, 0:1]
    npos = statm_ref[...][:, 0:1]
    locsum = statm_ref[...][:, 1:2]
    N = jnp.sum(npos)
    out0 = jnp.sum(locsum) / N
    out1 = jnp.sum(clsloss) / N
    iota_s = jax.lax.broadcasted_iota(jnp.int32, (1, 128), 1)
    out_ref[...] = jnp.where(iota_s == 0, out0, jnp.where(iota_s == 1, out1, 0.0))


def kernel(loc_pred, conf_pred, targets, anchors):
    B, P, C = conf_pred.shape
    NO = targets.shape[1]
    NR = (P + _NL - 1) // _NL
    PAD = NR * _NL

    anc = jnp.pad(anchors.T, ((0, 0), (0, PAD - P))).reshape(4, NR, _NL)
    loc4 = jnp.pad(jnp.transpose(loc_pred, (0, 2, 1)),
                   ((0, 0), (0, 0), (0, PAD - P))).reshape(B, 4, NR, _NL)

    lab_t, stats_m = pl.pallas_call(
        functools.partial(_match_body, P),
        grid=(B,),
        in_specs=[
            pl.BlockSpec((1, NO, 5), lambda b: (b, 0, 0)),
            pl.BlockSpec((4, NR, _NL), lambda b: (0, 0, 0)),
            pl.BlockSpec((1, 4, NR, _NL), lambda b: (b, 0, 0, 0)),
        ],
        out_specs=[
            pl.BlockSpec((1, NR, _NL), lambda b: (b, 0, 0)),
            pl.BlockSpec((1, 8, 1), lambda b: (b, 0, 0)),
        ],
        out_shape=[
            jax.ShapeDtypeStruct((B, NR, _NL), jnp.float32),
            jax.ShapeDtypeStruct((B, 8, 1), jnp.float32),
        ],
    )(targets, anc, loc4)

    lab = lab_t.reshape(B, PAD)[:, :P].reshape(B, P, 1)

    chunk = 4368
    nch = (P + chunk - 1) // chunk
    cls, stats_c = pl.pallas_call(
        functools.partial(_conf_body, P, chunk),
        grid=(B, nch),
        in_specs=[
            pl.BlockSpec((1, chunk, C), lambda b, c: (b, c, 0)),
            pl.BlockSpec((1, chunk, 1), lambda b, c: (b, c, 0)),
        ],
        out_specs=[
            pl.BlockSpec((1, chunk, 1), lambda b, c: (b, c, 0)),
            pl.BlockSpec((1, 8, 1), lambda b, c: (b, 0, 0)),
        ],
        out_shape=[
            jax.ShapeDtypeStruct((B, P, 1), jnp.float32),
            jax.ShapeDtypeStruct((B, 8, 1), jnp.float32),
        ],
    )(conf_pred, lab)

    Ppad = ((P + 127) // 128) * 128
    cls_p = jnp.pad(cls.reshape(B, P), ((0, 0), (0, Ppad - P)))
    statm16 = jnp.pad(stats_m.reshape(B, 8), ((0, 0), (0, 8)))
    statc16 = jnp.pad(stats_c.reshape(B, 8), ((0, 0), (0, 8)))

    mesh = plsc.VectorSubcoreMesh(core_axis_name="c", subcore_axis_name="s")
    sc_mine = functools.partial(
        pl.kernel,
        mesh=mesh,
        out_type=jax.ShapeDtypeStruct((B, _L), jnp.float32),
        scratch_types=[
            pltpu.VMEM((Ppad,), jnp.float32),
            pltpu.VMEM((Ppad,), jnp.int32),
            pltpu.VMEM((_L,), jnp.float32),
            pltpu.VMEM((_L,), jnp.float32),
            pltpu.VMEM((_L,), jnp.float32),
            pltpu.VMEM((2 * _L,), jnp.float32),
        ],
    )(functools.partial(_sc_mine_body, P, C))
    cls_bits = jax.lax.bitcast_convert_type(cls_p, jnp.int32)
    cls_part = sc_mine(cls_p, cls_bits, statm16, statc16)

    out = pl.pallas_call(
        _final_body,
        grid=(1,),
        in_specs=[
            pl.BlockSpec((B, _L), lambda i: (0, 0)),
            pl.BlockSpec((B, 8), lambda i: (0, 0)),
        ],
        out_specs=pl.BlockSpec((1, 128), lambda i: (0, 0)),
        out_shape=jax.ShapeDtypeStruct((1, 128), jnp.float32),
    )(cls_part, stats_m.reshape(B, 8))

    return out[0, 0], out[0, 1]


# speedup vs baseline: 1.2701x; 1.0488x over previous
import functools
import math

import jax
import jax.numpy as jnp
from jax import lax
from jax.experimental import pallas as pl
from jax.experimental.pallas import tpu as pltpu
from jax.experimental.pallas import tpu_sc as plsc

_NEGPOS_RATIO = 3.0
_THRESHOLD = 0.5
_NL = 128
_L = 16


def _smooth_l1_sum(d):
    ad = jnp.abs(d)
    return jnp.sum(jnp.where(ad < 1.0, 0.5 * ad * ad, ad - 0.5))


def _match_body(num_anchors, tgt_ref, anc_ref, loc_ref, lab_ref, stats_ref):
    P = num_anchors
    NO = tgt_ref.shape[1]
    NR = anc_ref.shape[1]

    ax = anc_ref[0]
    ay = anc_ref[1]
    aw = anc_ref[2]
    ah = anc_ref[3]
    ax2 = ax + aw
    ay2 = ay + ah
    area_a = (ax2 - ax) * (ay2 - ay)

    i0 = jax.lax.broadcasted_iota(jnp.int32, (NR, _NL), 0)
    i1 = jax.lax.broadcasted_iota(jnp.int32, (NR, _NL), 1)
    flat = i0 * _NL + i1
    valid = flat < P

    tgt = tgt_ref[0]
    tx_v = tgt[:, 0][None, :]
    ty_v = tgt[:, 1][None, :]
    tx2_v = tgt[:, 2][None, :]
    ty2_v = tgt[:, 3][None, :]
    tl_v = tgt[:, 4][None, :]
    tw_v = tx2_v - tx_v
    th_v = ty2_v - ty_v
    log_tw = jnp.log(tw_v)
    log_th = jnp.log(th_v)

    def _sc(a, t):
        return jnp.sum(a[0:1, t:t + 1])

    bto = jnp.full((NR, _NL), -1.0, jnp.float32)
    bti = jnp.zeros((NR, _NL), jnp.int32)
    bpis = []
    for t in range(NO):
        tx, ty, tx2, ty2 = (_sc(tx_v, t), _sc(ty_v, t), _sc(tx2_v, t), _sc(ty2_v, t))
        area_t = (tx2 - tx) * (ty2 - ty)
        w = jnp.clip(jnp.minimum(tx2, ax2) - jnp.maximum(tx, ax), 0.0, None)
        h = jnp.clip(jnp.minimum(ty2, ay2) - jnp.maximum(ty, ay), 0.0, None)
        inter = w * h
        ov = inter / (area_t + area_a - inter)
        upd = ov > bto
        bti = jnp.where(upd, t, bti)
        bto = jnp.where(upd, ov, bto)
        mx = jnp.max(jnp.where(valid, ov, -1.0))
        bpis.append(jnp.min(jnp.where((ov == mx) & valid, flat, P)))

    log_aw = jnp.log(aw)
    log_ah = jnp.log(ah)
    scat = [jnp.zeros((NR, _NL), jnp.float32) for _ in range(5)]
    for t in range(NO):
        m = flat == bpis[t]
        f = [(_sc(tx_v, t) - ax) / aw,
             (_sc(ty_v, t) - ay) / ah,
             _sc(log_tw, t) - log_aw,
             _sc(log_th, t) - log_ah,
             jnp.broadcast_to(_sc(tl_v, t), (NR, _NL))]
        scat = [jnp.where(m, f[j], scat[j]) for j in range(5)]

    g = [jnp.zeros((NR, _NL), jnp.float32) for _ in range(5)]
    for t in range(NO):
        tm = bti == t
        vals = [_sc(tx_v, t), _sc(ty_v, t), _sc(tw_v, t), _sc(th_v, t), _sc(tl_v, t)]
        g = [jnp.where(tm, vals[j], g[j]) for j in range(5)]

    second = [(g[0] - ax) / aw, (g[1] - ay) / ah,
              jnp.log(g[2]) - log_aw, jnp.log(g[3]) - log_ah]

    over = bto > _THRESHOLD
    lab = jnp.where(over, g[4], scat[4])
    lab = jnp.where(valid, lab, 0.0)
    pos = lab > 0.0

    locsum = jnp.float32(0.0)
    for j in range(4):
        locT_j = jnp.where(over, second[j], scat[j])
        locsum = locsum + _smooth_l1_sum(
            jnp.where(pos, loc_ref[0, j] - locT_j, 0.0))
    npos = jnp.sum(jnp.where(pos, 1.0, 0.0))

    lab_ref[0] = lab
    iota8 = jax.lax.broadcasted_iota(jnp.int32, (8, 1), 0)
    stats_ref[0] = jnp.where(iota8 == 0, npos,
                             jnp.where(iota8 == 1, locsum, 0.0))


def _conf_body(num_anchors, chunk, conf_ref, lab_ref, cls_ref, stats_ref):
    pc = pl.program_id(1)
    Pc = conf_ref.shape[1]
    C = conf_ref.shape[2]

    rows = pc * chunk + jax.lax.broadcasted_iota(jnp.int32, (Pc, 1), 0)
    valid = rows < num_anchors

    cf = conf_ref[0]
    lab = lab_ref[0]
    pos = valid & (lab > 0.0)
    posf = pos.astype(jnp.float32)

    cmax = jnp.max(cf, axis=1, keepdims=True)
    lse = jnp.log(jnp.sum(jnp.exp(cf - cmax), axis=1, keepdims=True)) + cmax
    c0 = cf[:, 0:1]
    iota_c = jax.lax.broadcasted_iota(jnp.int32, (Pc, C), 1)
    clab = jnp.sum(jnp.where(iota_c == lab.astype(jnp.int32), cf, 0.0),
                   axis=1, keepdims=True)

    cls_ref[0] = jnp.where(valid, (lse - c0) * (1.0 - posf), 0.0)
    possum = jnp.sum(jnp.where(pos, lse - clab, 0.0))

    iota8 = jax.lax.broadcasted_iota(jnp.int32, (8, 1), 0)
    col = jnp.where(iota8 == 0, possum, 0.0)

    @pl.when(pc == 0)
    def _():
        stats_ref[0] = col

    @pl.when(pc != 0)
    def _():
        stats_ref[0] = stats_ref[0] + col


def _sc_mine_body(num_anchors, num_classes, cls_hbm, bits_hbm, statm_hbm,
                  statc_hbm, out_hbm, row_v, rowi_v, stm_v, stc_v, outv, perm_v):
    P = num_anchors
    b = lax.axis_index("s") * 2 + lax.axis_index("c")
    pltpu.sync_copy(cls_hbm.at[b], row_v)
    pltpu.sync_copy(bits_hbm.at[b], rowi_v)
    pltpu.sync_copy(statm_hbm.at[b], stm_v)
    pltpu.sync_copy(statc_hbm.at[b], stc_v)

    iota = lax.broadcasted_iota(jnp.int32, (_L,), 0)
    lane0 = iota == 0
    zeros = jnp.zeros((_L,), jnp.float32)
    ones = jnp.ones((_L,), jnp.float32)

    def _lanetree(x, op):
        for sh in (8, 4, 2, 1):
            perm_v[pl.ds(0, _L)] = x
            perm_v[pl.ds(_L, _L)] = x
            x = op(x, perm_v[pl.ds(sh, _L)])
        return x

    npos = _lanetree(jnp.where(lane0, stm_v[...], zeros), jnp.add)
    k = jnp.minimum(_NEGPOS_RATIO * npos, float(P - 1))

    ngrp = row_v.shape[0] // (8 * _L)

    def count_ge(cand):
        def grp(gi, acc):
            for u in range(8):
                bits = rowi_v[pl.ds(gi * (8 * _L) + u * _L, _L)]
                acc = acc + jnp.where(bits >= cand, ones, zeros)
            return acc
        return _lanetree(lax.fori_loop(0, ngrp, grp, zeros), jnp.add)

    t = jnp.zeros((_L,), jnp.int32)
    for it in range(31):
        cand = t | jnp.full((_L,), 1 << (30 - it), jnp.int32)
        t = jnp.where(count_ge(cand) >= k, cand, t)

    def tail(gi, carry):
        sumv, cntv, vmaxv = carry
        for u in range(8):
            base = gi * (8 * _L) + u * _L
            bits = rowi_v[pl.ds(base, _L)]
            vals = row_v[pl.ds(base, _L)]
            gt = bits > t
            sumv = sumv + jnp.where(gt, vals, zeros)
            cntv = cntv + jnp.where(gt, ones, zeros)
            vmaxv = jnp.maximum(vmaxv, jnp.where(bits <= t, vals, zeros))
        return (sumv, cntv, vmaxv)

    sumv, cntv, vmaxv = lax.fori_loop(0, ngrp, tail, (zeros, zeros, zeros))
    sumgt = _lanetree(sumv, jnp.add)
    m = k - _lanetree(cntv, jnp.add)
    vf = _lanetree(vmaxv, jnp.maximum)
    negsum = sumgt + jnp.where(m > 0.0, m * vf, zeros)

    clsloss = (stc_v[...] + negsum
               + (float(P) - (npos + k)) * math.log(num_classes))
    outv[...] = jnp.where(lane0, clsloss, zeros)
    pltpu.sync_copy(outv, out_hbm.at[b])


def _final_body(cls_part_ref, statm_ref, out_ref):
    clsloss = cls_part_ref[...][:, 0:1]
    npos = statm_ref[...][:, 0:1]
    locsum = statm_ref[...][:, 1:2]
    N = jnp.sum(npos)
    out0 = jnp.sum(locsum) / N
    out1 = jnp.sum(clsloss) / N
    iota_s = jax.lax.broadcasted_iota(jnp.int32, (1, 128), 1)
    out_ref[...] = jnp.where(iota_s == 0, out0, jnp.where(iota_s == 1, out1, 0.0))


def kernel(loc_pred, conf_pred, targets, anchors):
    B, P, C = conf_pred.shape
    NO = targets.shape[1]
    NR = (P + _NL - 1) // _NL
    PAD = NR * _NL

    anc = jnp.pad(anchors.T, ((0, 0), (0, PAD - P))).reshape(4, NR, _NL)
    loc4 = jnp.pad(jnp.transpose(loc_pred, (0, 2, 1)),
                   ((0, 0), (0, 0), (0, PAD - P))).reshape(B, 4, NR, _NL)

    lab_t, stats_m = pl.pallas_call(
        functools.partial(_match_body, P),
        grid=(B,),
        in_specs=[
            pl.BlockSpec((1, NO, 5), lambda b: (b, 0, 0)),
            pl.BlockSpec((4, NR, _NL), lambda b: (0, 0, 0)),
            pl.BlockSpec((1, 4, NR, _NL), lambda b: (b, 0, 0, 0)),
        ],
        out_specs=[
            pl.BlockSpec((1, NR, _NL), lambda b: (b, 0, 0)),
            pl.BlockSpec((1, 8, 1), lambda b: (b, 0, 0)),
        ],
        out_shape=[
            jax.ShapeDtypeStruct((B, NR, _NL), jnp.float32),
            jax.ShapeDtypeStruct((B, 8, 1), jnp.float32),
        ],
    )(targets, anc, loc4)

    lab = lab_t.reshape(B, PAD)[:, :P].reshape(B, P, 1)

    chunk = 8736
    nch = (P + chunk - 1) // chunk
    cls, stats_c = pl.pallas_call(
        functools.partial(_conf_body, P, chunk),
        grid=(B, nch),
        in_specs=[
            pl.BlockSpec((1, chunk, C), lambda b, c: (b, c, 0)),
            pl.BlockSpec((1, chunk, 1), lambda b, c: (b, c, 0)),
        ],
        out_specs=[
            pl.BlockSpec((1, chunk, 1), lambda b, c: (b, c, 0)),
            pl.BlockSpec((1, 8, 1), lambda b, c: (b, 0, 0)),
        ],
        out_shape=[
            jax.ShapeDtypeStruct((B, P, 1), jnp.float32),
            jax.ShapeDtypeStruct((B, 8, 1), jnp.float32),
        ],
    )(conf_pred, lab)

    Ppad = ((P + 127) // 128) * 128
    cls_p = jnp.pad(cls.reshape(B, P), ((0, 0), (0, Ppad - P)))
    statm16 = jnp.pad(stats_m.reshape(B, 8), ((0, 0), (0, 8)))
    statc16 = jnp.pad(stats_c.reshape(B, 8), ((0, 0), (0, 8)))

    mesh = plsc.VectorSubcoreMesh(core_axis_name="c", subcore_axis_name="s")
    sc_mine = functools.partial(
        pl.kernel,
        mesh=mesh,
        out_type=jax.ShapeDtypeStruct((B, _L), jnp.float32),
        scratch_types=[
            pltpu.VMEM((Ppad,), jnp.float32),
            pltpu.VMEM((Ppad,), jnp.int32),
            pltpu.VMEM((_L,), jnp.float32),
            pltpu.VMEM((_L,), jnp.float32),
            pltpu.VMEM((_L,), jnp.float32),
            pltpu.VMEM((2 * _L,), jnp.float32),
        ],
    )(functools.partial(_sc_mine_body, P, C))
    cls_bits = jax.lax.bitcast_convert_type(cls_p, jnp.int32)
    cls_part = sc_mine(cls_p, cls_bits, statm16, statc16)

    out = pl.pallas_call(
        _final_body,
        grid=(1,),
        in_specs=[
            pl.BlockSpec((B, _L), lambda i: (0, 0)),
            pl.BlockSpec((B, 8), lambda i: (0, 0)),
        ],
        out_specs=pl.BlockSpec((1, 128), lambda i: (0, 0)),
        out_shape=jax.ShapeDtypeStruct((1, 128), jnp.float32),
    )(cls_part, stats_m.reshape(B, 8))

    return out[0, 0], out[0, 1]
